# baseline (speedup 1.0000x reference)
import functools

import jax
import jax.numpy as jnp
from jax import lax
from jax.experimental import pallas as pl
from jax.experimental.pallas import tpu as pltpu
from jax.experimental.pallas import tpu_sc as plsc

_STRIDE = 18


def kernel(x):
    B, T, H, W, C = x.shape
    n_out = (T + _STRIDE - 1) // _STRIDE
    sites = B * H * W

    info = plsc.get_sparse_core_info()
    num_cores = info.num_cores
    num_workers = num_cores * info.num_subcores
    sites_per_worker = sites // num_workers

    xt = jnp.transpose(x, (0, 2, 3, 1, 4)).reshape(sites, T, C)

    mesh = plsc.VectorSubcoreMesh(core_axis_name="c", subcore_axis_name="s")

    @functools.partial(
        pl.kernel,
        mesh=mesh,
        out_type=jax.ShapeDtypeStruct((sites, n_out, C), jnp.float32),
        scratch_types=[
            pltpu.VMEM((sites_per_worker, n_out, C), jnp.float32),
            pltpu.SemaphoreType.DMA,
            pltpu.SemaphoreType.DMA,
        ],
        compiler_params=pltpu.CompilerParams(
            use_tc_tiling_on_sc=True, skip_device_barrier=True
        ),
    )
    def gather_frames(x_hbm, out_hbm, buf, gsem, osem):
        wid = lax.axis_index("s") * num_cores + lax.axis_index("c")
        base = wid * sites_per_worker
        gathers = [
            pltpu.async_copy(
                x_hbm.at[pl.ds(base, sites_per_worker), i * _STRIDE],
                buf.at[:, i],
                gsem,
            )
            for i in range(n_out)
        ]
        writes = []
        for i, g in enumerate(gathers):
            g.wait()
            writes.append(
                pltpu.async_copy(
                    buf.at[:, i],
                    out_hbm.at[pl.ds(base, sites_per_worker), i],
                    osem,
                )
            )
        for w in writes:
            w.wait()

    out3 = gather_frames(xt)
    return jnp.transpose(out3.reshape(B, H, W, n_out, C), (0, 3, 1, 2, 4))


# speedup vs baseline: 1.0011x; 1.0011x over previous
import functools

import jax
import jax.numpy as jnp
from jax import lax
from jax.experimental import pallas as pl
from jax.experimental.pallas import tpu as pltpu
from jax.experimental.pallas import tpu_sc as plsc

_STRIDE = 18


def kernel(x):
    B, T, H, W, C = x.shape
    n_out = (T + _STRIDE - 1) // _STRIDE
    sites = B * H * W

    info = plsc.get_sparse_core_info()
    num_cores = info.num_cores
    num_workers = num_cores * info.num_subcores
    sites_per_worker = sites // num_workers

    xt = jnp.transpose(x, (0, 2, 3, 1, 4)).reshape(sites, T, C)

    mesh = plsc.VectorSubcoreMesh(core_axis_name="c", subcore_axis_name="s")
    half = sites_per_worker // 2

    @functools.partial(
        pl.kernel,
        mesh=mesh,
        out_type=jax.ShapeDtypeStruct((sites, n_out, C), jnp.float32),
        scratch_types=[
            pltpu.VMEM((sites_per_worker, n_out, C), jnp.float32),
            pltpu.SemaphoreType.DMA,
            pltpu.SemaphoreType.DMA,
        ],
        compiler_params=pltpu.CompilerParams(
            use_tc_tiling_on_sc=True, skip_device_barrier=True
        ),
    )
    def gather_frames(x_hbm, out_hbm, buf, gsem, osem):
        wid = lax.axis_index("s") * num_cores + lax.axis_index("c")
        base = wid * sites_per_worker
        halves = [(0, half), (half, sites_per_worker - half)]
        gathers = [
            [
                pltpu.async_copy(
                    x_hbm.at[pl.ds(base + lo, n), i * _STRIDE],
                    buf.at[pl.ds(lo, n), i],
                    gsem,
                )
                for i in range(n_out)
            ]
            for lo, n in halves
        ]
        writes = []
        for (lo, n), gs in zip(halves, gathers):
            for g in gs:
                g.wait()
            writes.append(
                pltpu.async_copy(
                    buf.at[pl.ds(lo, n)],
                    out_hbm.at[pl.ds(base + lo, n)],
                    osem,
                )
            )
        for w in writes:
            w.wait()

    out3 = gather_frames(xt)
    return jnp.transpose(out3.reshape(B, H, W, n_out, C), (0, 3, 1, 2, 4))


# speedup vs baseline: 1.0068x; 1.0056x over previous
import functools

import jax
import jax.numpy as jnp
from jax import lax
from jax.experimental import pallas as pl
from jax.experimental.pallas import tpu as pltpu
from jax.experimental.pallas import tpu_sc as plsc

_STRIDE = 18


def kernel(x):
    B, T, H, W, C = x.shape
    n_out = (T + _STRIDE - 1) // _STRIDE
    sites = B * H * W

    info = plsc.get_sparse_core_info()
    num_cores = info.num_cores
    num_workers = num_cores * info.num_subcores
    sites_per_worker = sites // num_workers

    xt = jnp.transpose(x, (0, 2, 3, 1, 4)).reshape(sites, T, C)

    mesh = plsc.VectorSubcoreMesh(core_axis_name="c", subcore_axis_name="s")

    @functools.partial(
        pl.kernel,
        mesh=mesh,
        out_type=jax.ShapeDtypeStruct((sites, n_out, C), jnp.float32),
        scratch_types=[
            pltpu.VMEM((sites_per_worker, n_out, C), jnp.float32),
            pltpu.SemaphoreType.DMA,
        ],
        compiler_params=pltpu.CompilerParams(use_tc_tiling_on_sc=True),
    )
    def gather_frames(x_hbm, out_hbm, buf, sem):
        wid = lax.axis_index("s") * num_cores + lax.axis_index("c")
        base = wid * sites_per_worker
        gathers = [
            pltpu.async_copy(
                x_hbm.at[pl.ds(base, sites_per_worker), i * _STRIDE],
                buf.at[:, i],
                sem,
            )
            for i in range(n_out)
        ]
        for g in gathers:
            g.wait()
        pltpu.sync_copy(buf, out_hbm.at[pl.ds(base, sites_per_worker)])

    out3 = gather_frames(xt)
    return jnp.transpose(out3.reshape(B, H, W, n_out, C), (0, 3, 1, 2, 4))
